# baseline (speedup 1.0000x reference)
import functools

import jax
import jax.numpy as jnp
from jax import lax
from jax.experimental import pallas as pl
from jax.experimental.pallas import tpu as pltpu
from jax.experimental.pallas import tpu_sc as plsc

_N = 10000
_E = 320000
_D = 128
_NC = 2
_NS = 16
_NW = _NC * _NS
_CH = 40
_NBUF = 4
_G0 = 128
_G1 = 0
_IR = 4
_EP = _NS * (_G0 + _G1) * _NBUF * _CH
_NP = 10240
_RPT = _NP // _NS
_ZR = _CH

_mesh = plsc.VectorSubcoreMesh(
    core_axis_name="c", subcore_axis_name="s", num_cores=_NC, num_subcores=_NS
)


@functools.partial(
    pl.kernel,
    out_type=jax.ShapeDtypeStruct((_NC, _NP, _D), jnp.float32),
    mesh=_mesh,
    scratch_types=[
        pltpu.VMEM((_IR, _NBUF, _CH), jnp.int32),
        pltpu.VMEM((_IR, _NBUF, _CH), jnp.int32),
        pltpu.VMEM((_NBUF, _CH, _D), jnp.float32),
        pltpu.VMEM((_NBUF, _CH, _D), jnp.float32),
        pltpu.VMEM_SHARED((_NP, _D), jnp.float32),
        pltpu.SemaphoreType.DMA,
        pltpu.SemaphoreType.DMA,
        pltpu.SemaphoreType.DMA,
    ],
)
def _sc_spmm(table_hbm, src_hbm, dst_hbm, out_hbm,
             src_v, dst_v, buf_a, buf_b, acc, gsem, ssem, isem):
    cid = lax.axis_index("c")
    sid = lax.axis_index("s")
    wid = cid * _NS + sid
    gc = jnp.where(cid == 0, _G0, _G1)

    base = sid * _RPT
    with jax.named_scope("zero_acc"):
        def _zero_row(r, carry):
            z = jnp.zeros((16,), jnp.float32)
            for cc in range(_D // 16):
                buf_a[0, r, pl.ds(cc * 16, 16)] = z
            return carry
        lax.fori_loop(0, _ZR, _zero_row, 0)
        zdescs = []
        for i in range(_RPT // _ZR):
            zdescs.append(pltpu.async_copy(
                buf_a.at[0], acc.at[pl.ds(base + i * _ZR, _ZR)], ssem))
        for d in zdescs:
            d.wait()
        plsc.subcore_barrier()

    def _fire_idx(g, s):
        pltpu.async_copy(src_hbm.at[wid, g], src_v.at[s], isem)
        pltpu.async_copy(dst_hbm.at[wid, g], dst_v.at[s], isem)

    def _drain_idx(g, s):
        pltpu.make_async_copy(src_hbm.at[wid, g], src_v.at[s], isem).wait()
        pltpu.make_async_copy(dst_hbm.at[wid, g], dst_v.at[s], isem).wait()

    def _fire_gathers(g, s, bufs):
        for b in range(_NBUF):
            pltpu.async_copy(table_hbm.at[src_v.at[s, b]], bufs.at[b], gsem)

    def _drain_gathers(g, s, bufs):
        for b in range(_NBUF):
            pltpu.make_async_copy(
                table_hbm.at[src_v.at[s, b]], bufs.at[b], gsem).wait()

    def _drain_scatters(g, s, bufs):
        for b in range(_NBUF):
            pltpu.make_async_copy(
                bufs.at[b], acc.at[dst_v.at[s, b]], ssem).wait()

    def _section(g, s, cur, other):
        s1 = (s + 1) % _IR
        s2 = (s + 2) % _IR
        sm = (s - 1) % _IR
        @pl.when(g + 2 < gc)
        def _():
            _fire_idx(g + 2, s2)
        @pl.when(g + 1 < gc)
        def _():
            _drain_idx(g + 1, s1)
        _drain_gathers(g, s, cur)
        @pl.when(g > 0)
        def _():
            _drain_scatters(g - 1, sm, other)
        @pl.when(g + 1 < gc)
        def _():
            _fire_gathers(g + 1, s1, other)
        for b in range(_NBUF):
            pltpu.async_copy(cur.at[b], acc.at[dst_v.at[s, b]], ssem, add=True)

    with jax.named_scope("edge_loop"):
        @pl.when(gc > 0)
        def _():
            _fire_idx(0, 0)
            _fire_idx(1, 1)
            _drain_idx(0, 0)
            _fire_gathers(0, 0, buf_a)

            def _quad(gg, carry):
                g = 4 * gg
                _section(g, 0, buf_a, buf_b)
                _section(g + 1, 1, buf_b, buf_a)
                _section(g + 2, 2, buf_a, buf_b)
                _section(g + 3, 3, buf_b, buf_a)
                return carry
            lax.fori_loop(0, gc // 4, _quad, 0)
            _drain_scatters(gc - 1, 3, buf_b)
        plsc.subcore_barrier()

    with jax.named_scope("writeout"):
        wdescs = []
        for i in range(_RPT // _ZR):
            sl = pl.ds(base + i * _ZR, _ZR)
            wdescs.append(pltpu.async_copy(acc.at[sl], out_hbm.at[cid].at[sl], gsem))
        for d in wdescs:
            d.wait()


_BLK = 1000


def _combine1_body(x_ref, p_ref, w_ref, b_ref, t1_ref, part_ref):
    t1 = p_ref[0] + p_ref[1]
    t1_ref[...] = t1
    x = x_ref[...]
    part = jnp.dot(x, w_ref[0], preferred_element_type=jnp.float32)
    part += jnp.dot(t1, w_ref[1], preferred_element_type=jnp.float32)
    part_ref[...] = part + b_ref[...]


def _final_body(x_ref, q_ref, w_ref, part_ref, out_ref):
    t2 = 2.0 * (q_ref[0] + q_ref[1]) - x_ref[...]
    out_ref[...] = part_ref[...] + jnp.dot(
        t2, w_ref[2], preferred_element_type=jnp.float32
    )


def kernel(x, edge_index, W, b):
    npad = _EP - _E
    n0 = _NS * _G0 * _NBUF * _CH

    def _split(flat):
        a = flat[:n0].reshape(_NS, _G0, _NBUF, _CH)
        z = flat[n0:].reshape(_NS, _G1, _NBUF, _CH)
        z = jnp.pad(z, ((0, 0), (0, _G0 - _G1), (0, 0), (0, 0)))
        return jnp.concatenate([a, z], axis=0)

    src3 = _split(jnp.concatenate(
        [edge_index[0], jnp.zeros((npad,), jnp.int32)]))
    dst3 = _split(jnp.concatenate(
        [edge_index[1], jnp.full((npad,), _N, jnp.int32)]))
    b2 = b.reshape(1, -1)

    p = _sc_spmm(x, src3, dst3)

    grid = _N // _BLK
    t1, part = pl.pallas_call(
        _combine1_body,
        grid=(grid,),
        in_specs=[
            pl.BlockSpec((_BLK, _D), lambda i: (i, 0)),
            pl.BlockSpec((_NC, _BLK, _D), lambda i: (0, i, 0)),
            pl.BlockSpec((3, _D, _D), lambda i: (0, 0, 0)),
            pl.BlockSpec((1, _D), lambda i: (0, 0)),
        ],
        out_specs=[
            pl.BlockSpec((_BLK, _D), lambda i: (i, 0)),
            pl.BlockSpec((_BLK, _D), lambda i: (i, 0)),
        ],
        out_shape=[
            jax.ShapeDtypeStruct((_N, _D), jnp.float32),
            jax.ShapeDtypeStruct((_N, _D), jnp.float32),
        ],
    )(x, p, W, b2)

    q = _sc_spmm(t1, src3, dst3)

    out = pl.pallas_call(
        _final_body,
        grid=(grid,),
        in_specs=[
            pl.BlockSpec((_BLK, _D), lambda i: (i, 0)),
            pl.BlockSpec((_NC, _BLK, _D), lambda i: (0, i, 0)),
            pl.BlockSpec((3, _D, _D), lambda i: (0, 0, 0)),
            pl.BlockSpec((_BLK, _D), lambda i: (i, 0)),
        ],
        out_specs=pl.BlockSpec((_BLK, _D), lambda i: (i, 0)),
        out_shape=jax.ShapeDtypeStruct((_N, _D), jnp.float32),
    )(x, q, W, part)

    return out


# speedup vs baseline: 3.9916x; 3.9916x over previous
import functools

import jax
import jax.numpy as jnp
from jax import lax
from jax.experimental import pallas as pl
from jax.experimental.pallas import tpu as pltpu
from jax.experimental.pallas import tpu_sc as plsc

_N = 10000
_E = 320000
_D = 128
_NC = 2
_NS = 16
_NW = _NC * _NS
_CH = 40
_NBUF = 4
_G0 = 64
_G1 = 64
_IR = 4
_EP = _NS * (_G0 + _G1) * _NBUF * _CH
_NP = 10240
_RPT = _NP // _NS
_ZR = _CH

_mesh = plsc.VectorSubcoreMesh(
    core_axis_name="c", subcore_axis_name="s", num_cores=_NC, num_subcores=_NS
)


@functools.partial(
    pl.kernel,
    out_type=jax.ShapeDtypeStruct((_NC, _NP, _D), jnp.float32),
    mesh=_mesh,
    scratch_types=[
        pltpu.VMEM((_IR, _NBUF, _CH), jnp.int32),
        pltpu.VMEM((_IR, _NBUF, _CH), jnp.int32),
        pltpu.VMEM((_NBUF, _CH, _D), jnp.float32),
        pltpu.VMEM((_NBUF, _CH, _D), jnp.float32),
        pltpu.VMEM_SHARED((_NP, _D), jnp.float32),
        pltpu.SemaphoreType.DMA,
        pltpu.SemaphoreType.DMA,
        pltpu.SemaphoreType.DMA,
    ],
)
def _sc_spmm(table_hbm, src_hbm, dst_hbm, out_hbm,
             src_v, dst_v, buf_a, buf_b, acc, gsem, ssem, isem):
    cid = lax.axis_index("c")
    sid = lax.axis_index("s")
    wid = cid * _NS + sid
    gc = jnp.where(cid == 0, _G0, _G1)

    base = sid * _RPT
    with jax.named_scope("zero_acc"):
        def _zero_row(r, carry):
            z = jnp.zeros((16,), jnp.float32)
            for cc in range(_D // 16):
                buf_a[0, r, pl.ds(cc * 16, 16)] = z
            return carry
        lax.fori_loop(0, _ZR, _zero_row, 0)
        zdescs = []
        for i in range(_RPT // _ZR):
            zdescs.append(pltpu.async_copy(
                buf_a.at[0], acc.at[pl.ds(base + i * _ZR, _ZR)], ssem))
        for d in zdescs:
            d.wait()
        plsc.subcore_barrier()

    def _fire_idx(g, s):
        pltpu.async_copy(src_hbm.at[wid, g], src_v.at[s], isem)
        pltpu.async_copy(dst_hbm.at[wid, g], dst_v.at[s], isem)

    def _drain_idx(g, s):
        pltpu.make_async_copy(src_hbm.at[wid, g], src_v.at[s], isem).wait()
        pltpu.make_async_copy(dst_hbm.at[wid, g], dst_v.at[s], isem).wait()

    def _fire_gathers(g, s, bufs):
        for b in range(_NBUF):
            pltpu.async_copy(table_hbm.at[src_v.at[s, b]], bufs.at[b], gsem)

    def _drain_gathers(g, s, bufs):
        for b in range(_NBUF):
            pltpu.make_async_copy(
                table_hbm.at[src_v.at[s, b]], bufs.at[b], gsem).wait()

    def _drain_scatters(g, s, bufs):
        for b in range(_NBUF):
            pltpu.make_async_copy(
                bufs.at[b], acc.at[dst_v.at[s, b]], ssem).wait()

    def _section(g, s, cur, other):
        s1 = (s + 1) % _IR
        s2 = (s + 2) % _IR
        sm = (s - 1) % _IR
        @pl.when(g + 2 < gc)
        def _():
            _fire_idx(g + 2, s2)
        @pl.when(g + 1 < gc)
        def _():
            _drain_idx(g + 1, s1)
        _drain_gathers(g, s, cur)
        @pl.when(g > 0)
        def _():
            _drain_scatters(g - 1, sm, other)
        @pl.when(g + 1 < gc)
        def _():
            _fire_gathers(g + 1, s1, other)
        for b in range(_NBUF):
            pltpu.async_copy(cur.at[b], acc.at[dst_v.at[s, b]], ssem, add=True)

    with jax.named_scope("edge_loop"):
        @pl.when(gc > 0)
        def _():
            _fire_idx(0, 0)
            _fire_idx(1, 1)
            _drain_idx(0, 0)
            _fire_gathers(0, 0, buf_a)

            def _quad(gg, carry):
                g = 4 * gg
                _section(g, 0, buf_a, buf_b)
                _section(g + 1, 1, buf_b, buf_a)
                _section(g + 2, 2, buf_a, buf_b)
                _section(g + 3, 3, buf_b, buf_a)
                return carry
            lax.fori_loop(0, gc // 4, _quad, 0)
            _drain_scatters(gc - 1, 3, buf_b)
        plsc.subcore_barrier()

    with jax.named_scope("writeout"):
        wdescs = []
        for i in range(_RPT // _ZR):
            sl = pl.ds(base + i * _ZR, _ZR)
            wdescs.append(pltpu.async_copy(acc.at[sl], out_hbm.at[cid].at[sl], gsem))
        for d in wdescs:
            d.wait()


_BLK = 1000


def _combine1_body(x_ref, p_ref, w_ref, b_ref, t1_ref, part_ref):
    t1 = p_ref[0] + p_ref[1]
    t1_ref[...] = t1
    x = x_ref[...]
    part = jnp.dot(x, w_ref[0], preferred_element_type=jnp.float32)
    part += jnp.dot(t1, w_ref[1], preferred_element_type=jnp.float32)
    part_ref[...] = part + b_ref[...]


def _final_body(x_ref, q_ref, w_ref, part_ref, out_ref):
    t2 = 2.0 * (q_ref[0] + q_ref[1]) - x_ref[...]
    out_ref[...] = part_ref[...] + jnp.dot(
        t2, w_ref[2], preferred_element_type=jnp.float32
    )


def kernel(x, edge_index, W, b):
    npad = _EP - _E
    n0 = _NS * _G0 * _NBUF * _CH

    def _split(flat):
        a = flat[:n0].reshape(_NS, _G0, _NBUF, _CH)
        z = flat[n0:].reshape(_NS, _G1, _NBUF, _CH)
        z = jnp.pad(z, ((0, 0), (0, _G0 - _G1), (0, 0), (0, 0)))
        return jnp.concatenate([a, z], axis=0)

    src3 = _split(jnp.concatenate(
        [edge_index[0], jnp.arange(npad, dtype=jnp.int32)]))
    dst3 = _split(jnp.concatenate(
        [edge_index[1], jnp.full((npad,), _N, jnp.int32)]))
    b2 = b.reshape(1, -1)

    p = _sc_spmm(x, src3, dst3)

    grid = _N // _BLK
    t1, part = pl.pallas_call(
        _combine1_body,
        grid=(grid,),
        in_specs=[
            pl.BlockSpec((_BLK, _D), lambda i: (i, 0)),
            pl.BlockSpec((_NC, _BLK, _D), lambda i: (0, i, 0)),
            pl.BlockSpec((3, _D, _D), lambda i: (0, 0, 0)),
            pl.BlockSpec((1, _D), lambda i: (0, 0)),
        ],
        out_specs=[
            pl.BlockSpec((_BLK, _D), lambda i: (i, 0)),
            pl.BlockSpec((_BLK, _D), lambda i: (i, 0)),
        ],
        out_shape=[
            jax.ShapeDtypeStruct((_N, _D), jnp.float32),
            jax.ShapeDtypeStruct((_N, _D), jnp.float32),
        ],
    )(x, p, W, b2)

    q = _sc_spmm(t1, src3, dst3)

    out = pl.pallas_call(
        _final_body,
        grid=(grid,),
        in_specs=[
            pl.BlockSpec((_BLK, _D), lambda i: (i, 0)),
            pl.BlockSpec((_NC, _BLK, _D), lambda i: (0, i, 0)),
            pl.BlockSpec((3, _D, _D), lambda i: (0, 0, 0)),
            pl.BlockSpec((_BLK, _D), lambda i: (i, 0)),
        ],
        out_specs=pl.BlockSpec((_BLK, _D), lambda i: (i, 0)),
        out_shape=jax.ShapeDtypeStruct((_N, _D), jnp.float32),
    )(x, q, W, part)

    return out
